# baseline (speedup 1.0000x reference)
import numpy as np
import jax
import jax.numpy as jnp
from jax.experimental import pallas as pl
from jax.experimental.pallas import tpu as pltpu

F_W, F_H, F_S = 50, 37, 16
SCALES = (8.0, 16.0, 24.0)
RATIOS = (0.5, 1.0, 2.0)
K1 = 400
NMS_THRESH = 0.6
BN_EPS = 1e-5

NA = F_H * F_W * 9
SPAD = 16896
SROWS, SCOLS = 132, 128
IMGS_PER_PROG = 4


def _build_anchors_np():
    anchors = []
    for y in range(F_H):
        for x in range(F_W):
            cx = x * F_S + F_S / 2.0
            cy = y * F_S + F_S / 2.0
            for r in RATIOS:
                for s in SCALES:
                    h = F_S * s * np.sqrt(r)
                    w = F_S * s / np.sqrt(r)
                    anchors.append([cx - w / 2, cy - h / 2, cx + w / 2, cy + h / 2])
    return np.asarray(anchors, dtype=np.float32)

_ANCH_NP = _build_anchors_np()
_VALID_NP = ((_ANCH_NP[:, 0] >= 0) & (_ANCH_NP[:, 1] >= 0) &
             (_ANCH_NP[:, 2] < F_W * F_S) & (_ANCH_NP[:, 3] < F_H * F_S))
_ANCH_PAD_NP = np.zeros((SPAD, 4), np.float32)
_ANCH_PAD_NP[:NA] = _ANCH_NP
_VMASK_PAD_NP = np.zeros((SPAD,), np.float32)
_VMASK_PAD_NP[:NA] = _VALID_NP.astype(np.float32)


def _conv(x, w, b, pad):
    y = jax.lax.conv_general_dilated(x, w, (1, 1), [(pad, pad), (pad, pad)],
                                     dimension_numbers=('NCHW', 'OIHW', 'NCHW'))
    return y + b[None, :, None, None]


def _rp_kernel(scores_in, off_in, anch_in, boxes_out, keep_out, sc_ref, ta_ref, iou_ref, sup_ref):
    idx2d = (jax.lax.broadcasted_iota(jnp.int32, (SROWS, SCOLS), 0) * SCOLS
             + jax.lax.broadcasted_iota(jnp.int32, (SROWS, SCOLS), 1))

    for img in range(IMGS_PER_PROG):
        sc_ref[img] = scores_in[img]
        sup_ref[img] = jnp.zeros((1, K1), jnp.float32)

    def sel_body(t, _):
        for img in range(IMGS_PER_PROG):
            s = sc_ref[img]
            m = jnp.max(s)
            j = jnp.min(jnp.where(s == m, idx2d, jnp.int32(SPAD)))
            sc_ref[img] = jnp.where(idx2d == j, jnp.float32(-2.0), s)
            r = j // SCOLS
            c = j - r * SCOLS
            offrows = jnp.reshape(off_in[img, :, pl.ds(r, 1), :], (4, SCOLS))
            anchrows = jnp.reshape(anch_in[:, pl.ds(r, 1), :], (4, SCOLS))
            rows8 = jnp.concatenate([offrows, anchrows], axis=0)
            picked = jnp.take_along_axis(rows8, jnp.full((8, 1), c, jnp.int32), axis=1)
            ta_ref[img, pl.ds(t, 1), :] = jnp.reshape(picked, (1, 8))
        return 0

    jax.lax.fori_loop(0, K1, sel_body, 0)

    for img in range(IMGS_PER_PROG):
        ta = ta_ref[img]
        t0, t1 = ta[:, 0:1], ta[:, 1:2]
        t2, t3 = ta[:, 2:3], ta[:, 3:4]
        a0, a1 = ta[:, 4:5], ta[:, 5:6]
        a2, a3 = ta[:, 6:7], ta[:, 7:8]
        xa = (a0 + a2) / 2
        ya = (a1 + a3) / 2
        wa = a2 - a0 + 1.0
        ha = a3 - a1 + 1.0
        x = t0 * wa + xa
        y = t1 * ha + ya
        w = wa * jnp.exp(t2)
        h = ha * jnp.exp(t3)
        bx1 = x - w / 2
        by1 = y - h / 2
        bx2 = x + w / 2
        by2 = y + h / 2
        boxes_out[img, :, 0:1] = bx1
        boxes_out[img, :, 1:2] = by1
        boxes_out[img, :, 2:3] = bx2
        boxes_out[img, :, 3:4] = by2
        x1, y1 = jnp.trunc(bx1), jnp.trunc(by1)
        x2, y2 = jnp.trunc(bx2), jnp.trunc(by2)
        area = (x2 - x1 + 1.0) * (y2 - y1 + 1.0)
        x1t = jnp.reshape(x1, (1, K1))
        x2t = jnp.reshape(x2, (1, K1))
        y1t = jnp.reshape(y1, (1, K1))
        y2t = jnp.reshape(y2, (1, K1))
        areat = jnp.reshape(area, (1, K1))
        iw = jnp.clip(jnp.minimum(x2, x2t) - jnp.maximum(x1, x1t) + 1.0, 0.0)
        ih = jnp.clip(jnp.minimum(y2, y2t) - jnp.maximum(y1, y1t) + 1.0, 0.0)
        inter = iw * ih
        iou_ref[img] = inter / (area + areat - inter)

    lane = jax.lax.broadcasted_iota(jnp.int32, (1, K1), 1)

    def nms_body(i, _):
        for img in range(IMGS_PER_PROG):
            sp = sup_ref[img]
            alive = 1.0 - jnp.max(jnp.where(lane == i, sp, jnp.float32(0.0)))
            row = iou_ref[img, pl.ds(i, 1), :]
            cand = jnp.where((row > NMS_THRESH) & (lane > i), jnp.float32(1.0), jnp.float32(0.0))
            sup_ref[img] = jnp.maximum(sp, cand * alive)
        return 0

    jax.lax.fori_loop(0, K1, nms_body, 0)

    for img in range(IMGS_PER_PROG):
        keep_out[img] = 1.0 - sup_ref[img]


def _run_rp(scores, offp, interpret=False):
    B = scores.shape[0]
    anch = jnp.asarray(_ANCH_PAD_NP.T.reshape(4, SROWS, SCOLS))
    grid = (B // IMGS_PER_PROG,)
    boxes, keepf = pl.pallas_call(
        _rp_kernel,
        grid=grid,
        in_specs=[
            pl.BlockSpec((IMGS_PER_PROG, SROWS, SCOLS), lambda g: (g, 0, 0)),
            pl.BlockSpec((IMGS_PER_PROG, 4, SROWS, SCOLS), lambda g: (g, 0, 0, 0)),
            pl.BlockSpec((4, SROWS, SCOLS), lambda g: (0, 0, 0)),
        ],
        out_specs=[
            pl.BlockSpec((IMGS_PER_PROG, K1, 4), lambda g: (g, 0, 0)),
            pl.BlockSpec((IMGS_PER_PROG, 1, K1), lambda g: (g, 0, 0)),
        ],
        out_shape=[
            jax.ShapeDtypeStruct((B, K1, 4), jnp.float32),
            jax.ShapeDtypeStruct((B, 1, K1), jnp.float32),
        ],
        scratch_shapes=[
            pltpu.VMEM((IMGS_PER_PROG, SROWS, SCOLS), jnp.float32),
            pltpu.VMEM((IMGS_PER_PROG, K1, 8), jnp.float32),
            pltpu.VMEM((IMGS_PER_PROG, K1, K1), jnp.float32),
            pltpu.VMEM((IMGS_PER_PROG, 1, K1), jnp.float32),
        ],
        compiler_params=pltpu.CompilerParams(dimension_semantics=("parallel",)),
        interpret=interpret,
    )(scores, offp, anch)
    return boxes, keepf


def kernel(feature, md_w, md_b, bn_gamma, bn_beta, cls_w, cls_b, off_w, off_b):
    B = feature.shape[0]
    out = _conv(feature, md_w, md_b, pad=1)
    mean = out.mean(axis=(0, 2, 3), keepdims=True)
    var = out.var(axis=(0, 2, 3), keepdims=True)
    out = bn_gamma[None, :, None, None] * (out - mean) * jax.lax.rsqrt(var + BN_EPS) + bn_beta[None, :, None, None]
    out = jax.nn.relu(out)
    cls = _conv(out, cls_w, cls_b, pad=0)
    off = _conv(out, off_w, off_b, pad=0)
    cls_pred = cls.transpose(0, 2, 3, 1).reshape(B, -1, 2)
    offset_pred = off.transpose(0, 2, 3, 1).reshape(B, -1, 4)

    score = jax.nn.softmax(cls_pred, axis=2)[:, :, 1]
    vmask = jnp.asarray(_VMASK_PAD_NP)[None, :NA]
    score = jnp.where(vmask > 0, score, -1.0)
    score = jnp.pad(score, ((0, 0), (0, SPAD - NA)), constant_values=-1.0)
    score = score.reshape(B, SROWS, SCOLS)
    offp = jnp.pad(offset_pred, ((0, 0), (0, SPAD - NA), (0, 0)))
    offp = offp.transpose(0, 2, 1).reshape(B, 4, SROWS, SCOLS)

    boxes, keepf = _run_rp(score, offp)
    keep = keepf[:, 0, :] > 0.5
    return offset_pred, cls_pred, boxes, keep
